# baseline (speedup 1.0000x reference)
import functools

import jax
import jax.numpy as jnp
from jax import lax
from jax.experimental import pallas as pl
from jax.experimental.pallas import tpu as pltpu
from jax.experimental.pallas import tpu_sc as plsc

_VOCAB = 1000000
_D = 64
_BATCH = 16384
_SEQ = 50
_N = _BATCH * _SEQ

_NC = 2
_NS = 16
_NW = _NC * _NS

_PER_W = _N // _NW
_CHUNK = 128
_NCHUNK = _PER_W // _CHUNK

_K = 4
_NBUF = 3
_NGROUP = _NCHUNK // _K


def _emb_body(idx_hbm, table_hbm, out_hbm, idx_v, rows_v, gsem, ssems):
    wid = lax.axis_index("s") * _NC + lax.axis_index("c")
    pltpu.sync_copy(idx_hbm.at[wid], idx_v)

    def fire_gathers(g, h):
        for b in range(_K):
            pltpu.async_copy(table_hbm.at[idx_v.at[g * _K + b]],
                             rows_v.at[h, b], gsem)

    def drain_gathers(h):
        for b in range(_K):
            pltpu.make_async_copy(table_hbm.at[pl.ds(0, _CHUNK)],
                                  rows_v.at[h, b], gsem).wait()

    def fire_stores(g, h):
        for b in range(_K):
            pltpu.async_copy(
                rows_v.at[h, b],
                out_hbm.at[wid, pl.ds((g * _K + b) * _CHUNK, _CHUNK)],
                ssems.at[h])

    def drain_stores(h):
        for b in range(_K):
            pltpu.make_async_copy(
                rows_v.at[h, b], out_hbm.at[wid, pl.ds(0, _CHUNK)],
                ssems.at[h]).wait()

    fire_gathers(0, 0)

    def body(g, _):
        h = lax.rem(g, _NBUF)
        drain_gathers(h)
        @pl.when(g + 1 < _NGROUP)
        def _():
            hn = lax.rem(g + 1, _NBUF)
            @pl.when(g + 1 >= _NBUF)
            def _():
                drain_stores(hn)
            fire_gathers(g + 1, hn)
        fire_stores(g, h)
        return 0

    lax.fori_loop(0, _NGROUP, body, 0)
    drain_stores((_NGROUP - 2) % _NBUF)
    drain_stores((_NGROUP - 1) % _NBUF)


_mesh = plsc.VectorSubcoreMesh(
    core_axis_name="c", subcore_axis_name="s",
    num_cores=_NC, num_subcores=_NS)

_emb = functools.partial(
    pl.kernel,
    out_type=jax.ShapeDtypeStruct((_NW, _PER_W, _D), jnp.float32),
    mesh=_mesh,
    scratch_types=[
        pltpu.VMEM((_NCHUNK, _CHUNK), jnp.int32),
        pltpu.VMEM((_NBUF, _K, _CHUNK, _D), jnp.float32),
        pltpu.SemaphoreType.DMA,
        pltpu.SemaphoreType.DMA((_NBUF,)),
    ],
    compiler_params=pltpu.CompilerParams(use_tc_tiling_on_sc=False),
)(_emb_body)


@jax.jit
def kernel(x, embed_weight):
    idx = x.reshape(_NW, _NCHUNK, _CHUNK).astype(jnp.int32)
    out = _emb(idx, embed_weight)
    return out.reshape(_BATCH, _SEQ, _D)
